# baseline (speedup 1.0000x reference)
import jax
import jax.numpy as jnp
import numpy as np
from jax.experimental import pallas as pl
from jax.experimental.pallas import tpu as pltpu

_B, _S, _D, _E = 1, 2048, 768, 8
_N = _B * _S
_TN = 512
_MAX_THRESHOLD = 0.1
_GCOLS = 16


def _moe_body(x_ref, wg_ref, bias_ref, eb_ref, ew_ref, out_ref):
    x = x_ref[...]
    logits = jnp.dot(x, wg_ref[...],
                     preferred_element_type=jnp.float32) + bias_ref[...]
    lt = logits.T
    g = lt[:_E, :]
    g = g - jnp.max(g, axis=0, keepdims=True)
    g = jnp.exp(g)
    g = g / jnp.sum(g, axis=0, keepdims=True)
    thr = jax.nn.sigmoid(lt[_E:_E + 1, :]) * _MAX_THRESHOLD
    ad = g - thr
    w = jnp.where(ad >= 0.0, ad, 0.0)
    s = jnp.sum(w, axis=0, keepdims=True)
    w = w / jnp.where(s == 0.0, 1.0, s)
    xw = jnp.concatenate(
        [w[e:e + 1, :].T * x for e in range(_E)], axis=1)
    acc = jnp.dot(xw, ew_ref[...], preferred_element_type=jnp.float32)
    out_ref[...] = acc + jax.lax.dot_general(
        w, eb_ref[...], (((0,), (0,)), ((), ())),
        preferred_element_type=jnp.float32)


def kernel(inputs, gate_W, gate_b, thr_W, thr_b, exp_W, exp_b):
    flat = inputs.reshape(_N, _D)
    wg = jnp.zeros((_D, _GCOLS), dtype=jnp.float32)
    wg = wg.at[:, :_E].set(gate_W).at[:, _E:_E + 1].set(thr_W)
    bias = jnp.zeros((1, _GCOLS), dtype=jnp.float32)
    bias = bias.at[:, :_E].set(gate_b[None, :]).at[:, _E].set(thr_b[0])
    ew = exp_W.reshape(_E * _D, _D)

    out = pl.pallas_call(
        _moe_body,
        grid=(_N // _TN,),
        in_specs=[
            pl.BlockSpec((_TN, _D), lambda i: (i, 0)),
            pl.BlockSpec((_D, _GCOLS), lambda i: (0, 0)),
            pl.BlockSpec((1, _GCOLS), lambda i: (0, 0)),
            pl.BlockSpec((_E, _D), lambda i: (0, 0)),
            pl.BlockSpec((_E * _D, _D), lambda i: (0, 0)),
        ],
        out_specs=pl.BlockSpec((_TN, _D), lambda i: (i, 0)),
        out_shape=jax.ShapeDtypeStruct((_N, _D), jnp.float32),
        compiler_params=pltpu.CompilerParams(
            dimension_semantics=("parallel",),
        ),
    )(flat, wg, bias, exp_b, ew)
    return out.reshape(inputs.shape[:-1] + (_D,))


# speedup vs baseline: 1.0250x; 1.0250x over previous
import jax
import jax.numpy as jnp
import numpy as np
from jax.experimental import pallas as pl
from jax.experimental.pallas import tpu as pltpu

_B, _S, _D, _E = 1, 2048, 768, 8
_N = _B * _S
_MAX_THRESHOLD = 0.1
_GCOLS = 16


def _moe_body(x_ref, wg_ref, bias_ref, ew_ref, out_ref, wt_scr):
    s = pl.program_id(0)

    @pl.when(s == 0)
    def _routing():
        logits = jnp.dot(x_ref[...], wg_ref[...],
                         preferred_element_type=jnp.float32) + bias_ref[...]
        lt = logits.T
        g = lt[:_E, :]
        g = g - jnp.max(g, axis=0, keepdims=True)
        g = jnp.exp(g)
        g = g / jnp.sum(g, axis=0, keepdims=True)
        thr = jax.nn.sigmoid(lt[_E:_E + 1, :]) * _MAX_THRESHOLD
        ad = g - thr
        w = jnp.where(ad >= 0.0, ad, 0.0)
        sw = jnp.sum(w, axis=0, keepdims=True)
        w = w / jnp.where(sw == 0.0, 1.0, sw)
        wt_scr[...] = w

    @pl.when(s > 0)
    def _expert():
        acc = jnp.dot(x_ref[...], ew_ref[0],
                      preferred_element_type=jnp.float32)
        wcol = wt_scr[pl.ds(s - 1, 1), :].T

        @pl.when(s == 1)
        def _():
            out_ref[...] = wcol * acc

        @pl.when(s > 1)
        def _():
            out_ref[...] += wcol * acc


def kernel(inputs, gate_W, gate_b, thr_W, thr_b, exp_W, exp_b):
    flat = inputs.reshape(_N, _D)
    wg = jnp.zeros((_D, _GCOLS), dtype=jnp.float32)
    wg = wg.at[:, :_E].set(gate_W).at[:, _E:_E + 1].set(thr_W)
    bias = jnp.zeros((1, _GCOLS), dtype=jnp.float32)
    bias = bias.at[:, :_E].set(gate_b[None, :]).at[:, _E].set(thr_b[0])

    out = pl.pallas_call(
        _moe_body,
        grid=(_E + 1,),
        in_specs=[
            pl.BlockSpec((_N, _D), lambda s: (0, 0)),
            pl.BlockSpec((_D, _GCOLS), lambda s: (0, 0)),
            pl.BlockSpec((1, _GCOLS), lambda s: (0, 0)),
            pl.BlockSpec((1, _D, _D), lambda s: (jnp.maximum(s - 1, 0), 0, 0)),
        ],
        out_specs=pl.BlockSpec((_N, _D), lambda s: (0, 0)),
        out_shape=jax.ShapeDtypeStruct((_N, _D), jnp.float32),
        scratch_shapes=[pltpu.VMEM((_E, _N), jnp.float32)],
        compiler_params=pltpu.CompilerParams(
            dimension_semantics=("arbitrary",),
        ),
    )(flat, wg, bias, exp_W)
    return out.reshape(inputs.shape[:-1] + (_D,))


# speedup vs baseline: 1.1027x; 1.0758x over previous
import jax
import jax.numpy as jnp
import numpy as np
from jax.experimental import pallas as pl
from jax.experimental.pallas import tpu as pltpu

_B, _S, _D, _E = 1, 2048, 768, 8
_N = _B * _S
_MAX_THRESHOLD = 0.1
_GCOLS = 16


def _moe_body(x_ref, wg_ref, bias_ref, ew_ref, out_ref, wt_scr):
    s = pl.program_id(0)

    @pl.when(s == 0)
    def _routing():
        logits = jnp.dot(x_ref[...], wg_ref[...],
                         preferred_element_type=jnp.float32) + bias_ref[...]
        lt = logits.T
        g = lt[:_E, :]
        g = g - jnp.max(g, axis=0, keepdims=True)
        g = jnp.exp(g)
        g = g / jnp.sum(g, axis=0, keepdims=True)
        thr = jax.nn.sigmoid(lt[_E:_E + 1, :]) * _MAX_THRESHOLD
        ad = g - thr
        w = jnp.where(ad >= 0.0, ad, 0.0)
        sw = jnp.sum(w, axis=0, keepdims=True)
        w = w / jnp.where(sw == 0.0, 1.0, sw)
        wt_scr[...] = w
        out_ref[...] = jnp.zeros((_N, _D), jnp.float32)

    @pl.when(s > 0)
    def _experts():
        e = 2 * (s - 1)
        acc0 = jnp.dot(x_ref[...], ew_ref[0],
                       preferred_element_type=jnp.float32)
        acc1 = jnp.dot(x_ref[...], ew_ref[1],
                       preferred_element_type=jnp.float32)
        wc0 = wt_scr[pl.ds(e, 1), :].T
        wc1 = wt_scr[pl.ds(e + 1, 1), :].T
        out_ref[...] += wc0 * acc0 + wc1 * acc1


def kernel(inputs, gate_W, gate_b, thr_W, thr_b, exp_W, exp_b):
    flat = inputs.reshape(_N, _D)
    wg = jnp.zeros((_D, _GCOLS), dtype=jnp.float32)
    wg = wg.at[:, :_E].set(gate_W).at[:, _E:_E + 1].set(thr_W)
    bias = jnp.zeros((1, _GCOLS), dtype=jnp.float32)
    bias = bias.at[:, :_E].set(gate_b[None, :]).at[:, _E].set(thr_b[0])

    out = pl.pallas_call(
        _moe_body,
        grid=(_E // 2 + 1,),
        in_specs=[
            pl.BlockSpec((_N, _D), lambda s: (0, 0)),
            pl.BlockSpec((_D, _GCOLS), lambda s: (0, 0)),
            pl.BlockSpec((1, _GCOLS), lambda s: (0, 0)),
            pl.BlockSpec((2, _D, _D), lambda s: (jnp.maximum(s - 1, 0), 0, 0)),
        ],
        out_specs=pl.BlockSpec((_N, _D), lambda s: (0, 0)),
        out_shape=jax.ShapeDtypeStruct((_N, _D), jnp.float32),
        scratch_shapes=[pltpu.VMEM((_E, _N), jnp.float32)],
        compiler_params=pltpu.CompilerParams(
            dimension_semantics=("arbitrary",),
        ),
    )(flat, wg, bias, exp_W)
    return out.reshape(inputs.shape[:-1] + (_D,))
